# Initial kernel scaffold; baseline (speedup 1.0000x reference)
import jax
import jax.numpy as jnp
from jax.experimental import pallas as pl


def kernel(y, neighbors_index, neighbors_row_splits, f_y, W1, b1, W2, b2, W3, b3):
    raise NotImplementedError("write your pallas kernel here")


# speedup vs baseline: 1.7797x; 1.7797x over previous
import functools

import jax
import jax.numpy as jnp
from jax import lax
from jax.experimental import pallas as pl
from jax.experimental.pallas import tpu as pltpu
from jax.experimental.pallas import tpu_sc as plsc

N = 50000
E = 800000
FCH = 16

NC = 2
NS = 16
NW = NC * NS

CH = 128
CHUNKS = 196
EW = CH * CHUNKS
E_PAD = EW * NW
E4 = E_PAD // 4
E8 = E_PAD // 8

P = 4
BT = 2048
NBLK = E8 // BT

NP = 50176
TSTRIPE = NP // NS

ACC_STRIPE = 3136
ACC_R = ACC_STRIPE * NS

BN = 5000


def _gather_body(ytab_hbm, ftab_hbm, idx_hbm, seg_hbm, x_hbm, fj8_hbm,
                 stage_v, idx_v, seg_v, yj_v, yi_v, fj_v,
                 ytab_s, sem1, sem2, sem3):
    cid = lax.axis_index("c")
    sid = lax.axis_index("s")
    wid = sid * NC + cid

    pltpu.sync_copy(ytab_hbm.at[pl.ds(sid * TSTRIPE, TSTRIPE)], stage_v)
    pltpu.sync_copy(stage_v, ytab_s.at[pl.ds(sid * TSTRIPE, TSTRIPE)])
    plsc.subcore_barrier()

    g = wid // 8
    rb = (wid % 8) * EW
    h = (wid % 8) // 4
    b8 = (wid % 4) * EW
    colj = 16 * g
    coli = 64 + 16 * g
    colf = 64 * h + 16 * g

    def body(ci, _):
        base = wid * EW + ci * CH
        r0 = rb + ci * CH
        r8 = b8 + ci * CH
        pltpu.sync_copy(idx_hbm.at[pl.ds(base, CH)], idx_v)
        pltpu.sync_copy(seg_hbm.at[pl.ds(base, CH)], seg_v)
        cp1 = pltpu.async_copy(ytab_s.at[idx_v], yj_v, sem1)
        cp2 = pltpu.async_copy(ytab_s.at[seg_v], yi_v, sem2)
        cp3 = pltpu.async_copy(ftab_hbm.at[idx_v], fj_v, sem3)
        cp1.wait()
        cp2.wait()
        cp3.wait()
        pltpu.sync_copy(yj_v, x_hbm.at[pl.ds(r0, CH), pl.ds(colj, 16)])
        pltpu.sync_copy(yi_v, x_hbm.at[pl.ds(r0, CH), pl.ds(coli, 16)])
        pltpu.sync_copy(fj_v, fj8_hbm.at[pl.ds(r8, CH), pl.ds(colf, 16)])
        return 0

    lax.fori_loop(0, CHUNKS, body, 0)


def _scatter_body(kk8_hbm, seg_hbm, zeros_hbm, part_hbm,
                  k_v, seg_v, acc):
    cid = lax.axis_index("c")
    sid = lax.axis_index("s")
    wid = sid * NC + cid

    pltpu.sync_copy(zeros_hbm, acc.at[pl.ds(sid * ACC_STRIPE, ACC_STRIPE)])
    plsc.subcore_barrier()

    g = wid // 8
    h = (wid % 8) // 4
    b8 = (wid % 4) * EW
    colf = 64 * h + 16 * g

    def body(ci, _):
        base = wid * EW + ci * CH
        r8 = b8 + ci * CH
        pltpu.sync_copy(seg_hbm.at[pl.ds(base, CH)], seg_v)
        pltpu.sync_copy(kk8_hbm.at[pl.ds(r8, CH), pl.ds(colf, 16)], k_v)
        pltpu.sync_copy(k_v, acc.at[seg_v], add=True)
        return 0

    lax.fori_loop(0, CHUNKS, body, 0)
    plsc.subcore_barrier()

    pltpu.sync_copy(acc.at[pl.ds(sid * ACC_STRIPE, ACC_STRIPE)],
                    part_hbm.at[cid, pl.ds(sid * ACC_STRIPE, ACC_STRIPE)])


def _mlp_body(x1_ref, x2_ref, fj_ref, w1_ref, b1_ref, w2_ref, b2_ref,
              w3_ref, b3_ref, out_ref):
    def mlp(x):
        h = jax.nn.gelu(jnp.dot(x, w1_ref[...],
                                preferred_element_type=jnp.float32)
                        + b1_ref[...])
        h = jax.nn.gelu(jnp.dot(h, w2_ref[...],
                                preferred_element_type=jnp.float32)
                        + b2_ref[...])
        return jnp.dot(h, w3_ref[...],
                       preferred_element_type=jnp.float32) + b3_ref[...]

    kk = jnp.concatenate([mlp(x1_ref[...]), mlp(x2_ref[...])], axis=1)
    out_ref[...] = kk * fj_ref[...]


def _combine_body(p0_ref, p1_ref, cnt_ref, out_ref):
    inv = 1.0 / jnp.maximum(cnt_ref[...], 1.0)
    out_ref[...] = (p0_ref[0] + p1_ref[0]) * inv


def kernel(y, neighbors_index, neighbors_row_splits, f_y, W1, b1, W2, b2, W3, b3):
    rs = neighbors_row_splits.astype(jnp.int32)
    counts = rs[1:] - rs[:-1]
    seg = jnp.repeat(jnp.arange(N, dtype=jnp.int32), counts,
                     total_repeat_length=E)
    idx_p = jnp.concatenate(
        [neighbors_index.astype(jnp.int32),
         jnp.zeros((E_PAD - E,), jnp.int32)])
    seg_p = jnp.concatenate([seg, jnp.full((E_PAD - E,), N, jnp.int32)])
    ytab = jnp.zeros((NP, 16), jnp.float32).at[:N, :3].set(y)
    ftab = jnp.zeros((NP, 16), jnp.float32).at[:N].set(f_y)
    zeros_stripe = jnp.zeros((ACC_STRIPE, 16), jnp.float32)
    cnt_f = counts.astype(jnp.float32).reshape(N, 1)

    w1j = jnp.zeros((16, 64), jnp.float32).at[0:3].set(W1[0:3])
    w1i = jnp.zeros((16, 64), jnp.float32).at[0:3].set(W1[3:6])
    w1p = jnp.zeros((128, 256), jnp.float32)
    w2d = jnp.zeros((256, 256), jnp.float32)
    w3d = jnp.zeros((256, 64), jnp.float32)
    for g in range(P):
        w1p = w1p.at[16 * g:16 * g + 16, 64 * g:64 * g + 64].set(w1j)
        w1p = w1p.at[64 + 16 * g:64 + 16 * g + 16,
                     64 * g:64 * g + 64].set(w1i)
        w2d = w2d.at[64 * g:64 * g + 64, 64 * g:64 * g + 64].set(W2)
        w3d = w3d.at[64 * g:64 * g + 64, 16 * g:16 * g + 16].set(W3)
    b1d = jnp.tile(b1, P).reshape(1, 256)
    b2d = jnp.tile(b2, P).reshape(1, 256)
    b3d = jnp.tile(b3, P).reshape(1, 64)

    mesh = plsc.VectorSubcoreMesh(core_axis_name="c", subcore_axis_name="s")
    sc_params = pltpu.CompilerParams(use_tc_tiling_on_sc=False)

    gather_k = pl.kernel(
        _gather_body,
        out_type=(jax.ShapeDtypeStruct((E4, 128), jnp.float32),
                  jax.ShapeDtypeStruct((E8, 128), jnp.float32)),
        mesh=mesh,
        compiler_params=sc_params,
        scratch_types=(
            pltpu.VMEM((TSTRIPE, 16), jnp.float32),
            pltpu.VMEM((CH,), jnp.int32),
            pltpu.VMEM((CH,), jnp.int32),
            pltpu.VMEM((CH, 16), jnp.float32),
            pltpu.VMEM((CH, 16), jnp.float32),
            pltpu.VMEM((CH, 16), jnp.float32),
            pltpu.MemorySpace.VMEM_SHARED((NP, 16), jnp.float32),
            pltpu.SemaphoreType.DMA,
            pltpu.SemaphoreType.DMA,
            pltpu.SemaphoreType.DMA,
        ),
    )
    x_arr, fj8 = gather_k(ytab, ftab, idx_p, seg_p)

    kk8 = pl.pallas_call(
        _mlp_body,
        grid=(NBLK,),
        in_specs=[
            pl.BlockSpec((BT, 128), lambda i: (i, 0)),
            pl.BlockSpec((BT, 128), lambda i: (i + NBLK, 0)),
            pl.BlockSpec((BT, 128), lambda i: (i, 0)),
            pl.BlockSpec((128, 256), lambda i: (0, 0)),
            pl.BlockSpec((1, 256), lambda i: (0, 0)),
            pl.BlockSpec((256, 256), lambda i: (0, 0)),
            pl.BlockSpec((1, 256), lambda i: (0, 0)),
            pl.BlockSpec((256, 64), lambda i: (0, 0)),
            pl.BlockSpec((1, 64), lambda i: (0, 0)),
        ],
        out_specs=pl.BlockSpec((BT, 128), lambda i: (i, 0)),
        out_shape=jax.ShapeDtypeStruct((E8, 128), jnp.float32),
    )(x_arr, x_arr, fj8, w1p, b1d, w2d, b2d, w3d, b3d)

    scatter_k = pl.kernel(
        _scatter_body,
        out_type=jax.ShapeDtypeStruct((NC, ACC_R, 16), jnp.float32),
        mesh=mesh,
        compiler_params=sc_params,
        scratch_types=(
            pltpu.VMEM((CH, 16), jnp.float32),
            pltpu.VMEM((CH,), jnp.int32),
            pltpu.MemorySpace.VMEM_SHARED((ACC_R, 16), jnp.float32),
        ),
    )
    partials = scatter_k(kk8, seg_p, zeros_stripe)

    out = pl.pallas_call(
        _combine_body,
        grid=(N // BN,),
        in_specs=[
            pl.BlockSpec((1, BN, 16), lambda i: (0, i, 0)),
            pl.BlockSpec((1, BN, 16), lambda i: (1, i, 0)),
            pl.BlockSpec((BN, 1), lambda i: (i, 0)),
        ],
        out_specs=pl.BlockSpec((BN, 16), lambda i: (i, 0)),
        out_shape=jax.ShapeDtypeStruct((N, FCH), jnp.float32),
    )(partials, partials, cnt_f)
    return out


# speedup vs baseline: 7.9648x; 4.4755x over previous
import functools

import jax
import jax.numpy as jnp
from jax import lax
from jax.experimental import pallas as pl
from jax.experimental.pallas import tpu as pltpu
from jax.experimental.pallas import tpu_sc as plsc

N = 50000
E = 800000
FCH = 16

NC = 2
NS = 16
NW = NC * NS

CH = 128
CHUNKS = 196
EW = CH * CHUNKS
E_PAD = EW * NW
E4 = E_PAD // 4
E8 = E_PAD // 8

P = 4
BT = 2048
NBLK = E8 // BT

NP = 50176
TSTRIPE = NP // NS

ACC_STRIPE = 3136
ACC_R = ACC_STRIPE * NS

BN = 5000


def _gather_body(ytab_hbm, ftab_hbm, idx_hbm, seg_hbm, x_hbm, fj8_hbm,
                 stage_v, idx_v, seg_v, yj_v, yi_v, fj_v,
                 ytab_s, sem1, sem2, sem3):
    cid = lax.axis_index("c")
    sid = lax.axis_index("s")
    wid = sid * NC + cid

    pltpu.sync_copy(ytab_hbm.at[pl.ds(sid * TSTRIPE, TSTRIPE)], stage_v)
    pltpu.sync_copy(stage_v, ytab_s.at[pl.ds(sid * TSTRIPE, TSTRIPE)])
    plsc.subcore_barrier()

    g = wid // 8
    rb = (wid % 8) * EW
    h = (wid % 8) // 4
    b8 = (wid % 4) * EW
    colj = 16 * g
    coli = 64 + 16 * g
    colf = 64 * h + 16 * g

    def body(ci, _):
        base = wid * EW + ci * CH
        r0 = rb + ci * CH
        r8 = b8 + ci * CH
        pltpu.sync_copy(idx_hbm.at[pl.ds(base, CH)], idx_v)
        pltpu.sync_copy(seg_hbm.at[pl.ds(base, CH)], seg_v)
        cp1 = pltpu.async_copy(ytab_s.at[idx_v], yj_v, sem1)
        cp2 = pltpu.async_copy(ytab_s.at[seg_v], yi_v, sem2)
        cp3 = pltpu.async_copy(ftab_hbm.at[idx_v], fj_v, sem3)
        cp1.wait()
        cp2.wait()
        cp3.wait()
        pltpu.sync_copy(yj_v, x_hbm.at[pl.ds(r0, CH), pl.ds(colj, 16)])
        pltpu.sync_copy(yi_v, x_hbm.at[pl.ds(r0, CH), pl.ds(coli, 16)])
        pltpu.sync_copy(fj_v, fj8_hbm.at[pl.ds(r8, CH), pl.ds(colf, 16)])
        return 0

    lax.fori_loop(0, CHUNKS, body, 0)


def _scatter_body(kk8_hbm, seg_hbm, zeros_hbm, part_hbm,
                  k_v, seg_v, acc):
    cid = lax.axis_index("c")
    sid = lax.axis_index("s")
    wid = sid * NC + cid

    pltpu.sync_copy(zeros_hbm, acc.at[pl.ds(sid * ACC_STRIPE, ACC_STRIPE)])
    plsc.subcore_barrier()

    g = wid // 8
    h = (wid % 8) // 4
    b8 = (wid % 4) * EW
    colf = 64 * h + 16 * g

    def body(ci, _):
        base = wid * EW + ci * CH
        r8 = b8 + ci * CH
        pltpu.sync_copy(seg_hbm.at[pl.ds(base, CH)], seg_v)
        pltpu.sync_copy(kk8_hbm.at[pl.ds(r8, CH), pl.ds(colf, 16)], k_v)
        pltpu.sync_copy(k_v, acc.at[seg_v], add=True)
        return 0

    lax.fori_loop(0, CHUNKS, body, 0)
    plsc.subcore_barrier()

    pltpu.sync_copy(acc.at[pl.ds(sid * ACC_STRIPE, ACC_STRIPE)],
                    part_hbm.at[cid, pl.ds(sid * ACC_STRIPE, ACC_STRIPE)])


def _mlp_body(x1_ref, x2_ref, fj_ref, w1_ref, b1_ref, w2_ref, b2_ref,
              w3_ref, b3_ref, out_ref):
    def mlp(x):
        h = jax.nn.gelu(jnp.dot(x, w1_ref[...],
                                preferred_element_type=jnp.float32)
                        + b1_ref[...])
        h = jax.nn.gelu(jnp.dot(h, w2_ref[...],
                                preferred_element_type=jnp.float32)
                        + b2_ref[...])
        return jnp.dot(h, w3_ref[...],
                       preferred_element_type=jnp.float32) + b3_ref[...]

    kk = jnp.concatenate([mlp(x1_ref[...]), mlp(x2_ref[...])], axis=1)
    out_ref[...] = kk * fj_ref[...]


def _combine_body(p0_ref, p1_ref, cnt_ref, out_ref):
    inv = 1.0 / jnp.maximum(cnt_ref[...], 1.0)
    out_ref[...] = (p0_ref[0] + p1_ref[0]) * inv


def kernel(y, neighbors_index, neighbors_row_splits, f_y, W1, b1, W2, b2, W3, b3):
    rs = neighbors_row_splits.astype(jnp.int32)
    counts = rs[1:] - rs[:-1]
    mark = jnp.zeros((E,), jnp.int32).at[rs[1:N]].add(1, mode="drop")
    seg = jnp.cumsum(mark)
    idx_p = jnp.concatenate(
        [neighbors_index.astype(jnp.int32),
         jnp.zeros((E_PAD - E,), jnp.int32)])
    seg_p = jnp.concatenate([seg, jnp.full((E_PAD - E,), N, jnp.int32)])
    ytab = jnp.zeros((NP, 16), jnp.float32).at[:N, :3].set(y)
    ftab = jnp.zeros((NP, 16), jnp.float32).at[:N].set(f_y)
    zeros_stripe = jnp.zeros((ACC_STRIPE, 16), jnp.float32)
    cnt_f = counts.astype(jnp.float32).reshape(N, 1)

    w1j = jnp.zeros((16, 64), jnp.float32).at[0:3].set(W1[0:3])
    w1i = jnp.zeros((16, 64), jnp.float32).at[0:3].set(W1[3:6])
    w1p = jnp.zeros((128, 256), jnp.float32)
    w2d = jnp.zeros((256, 256), jnp.float32)
    w3d = jnp.zeros((256, 64), jnp.float32)
    for g in range(P):
        w1p = w1p.at[16 * g:16 * g + 16, 64 * g:64 * g + 64].set(w1j)
        w1p = w1p.at[64 + 16 * g:64 + 16 * g + 16,
                     64 * g:64 * g + 64].set(w1i)
        w2d = w2d.at[64 * g:64 * g + 64, 64 * g:64 * g + 64].set(W2)
        w3d = w3d.at[64 * g:64 * g + 64, 16 * g:16 * g + 16].set(W3)
    b1d = jnp.tile(b1, P).reshape(1, 256)
    b2d = jnp.tile(b2, P).reshape(1, 256)
    b3d = jnp.tile(b3, P).reshape(1, 64)

    mesh = plsc.VectorSubcoreMesh(core_axis_name="c", subcore_axis_name="s")
    sc_params = pltpu.CompilerParams(use_tc_tiling_on_sc=False)

    gather_k = pl.kernel(
        _gather_body,
        out_type=(jax.ShapeDtypeStruct((E4, 128), jnp.float32),
                  jax.ShapeDtypeStruct((E8, 128), jnp.float32)),
        mesh=mesh,
        compiler_params=sc_params,
        scratch_types=(
            pltpu.VMEM((TSTRIPE, 16), jnp.float32),
            pltpu.VMEM((CH,), jnp.int32),
            pltpu.VMEM((CH,), jnp.int32),
            pltpu.VMEM((CH, 16), jnp.float32),
            pltpu.VMEM((CH, 16), jnp.float32),
            pltpu.VMEM((CH, 16), jnp.float32),
            pltpu.MemorySpace.VMEM_SHARED((NP, 16), jnp.float32),
            pltpu.SemaphoreType.DMA,
            pltpu.SemaphoreType.DMA,
            pltpu.SemaphoreType.DMA,
        ),
    )
    x_arr, fj8 = gather_k(ytab, ftab, idx_p, seg_p)

    kk8 = pl.pallas_call(
        _mlp_body,
        grid=(NBLK,),
        in_specs=[
            pl.BlockSpec((BT, 128), lambda i: (i, 0)),
            pl.BlockSpec((BT, 128), lambda i: (i + NBLK, 0)),
            pl.BlockSpec((BT, 128), lambda i: (i, 0)),
            pl.BlockSpec((128, 256), lambda i: (0, 0)),
            pl.BlockSpec((1, 256), lambda i: (0, 0)),
            pl.BlockSpec((256, 256), lambda i: (0, 0)),
            pl.BlockSpec((1, 256), lambda i: (0, 0)),
            pl.BlockSpec((256, 64), lambda i: (0, 0)),
            pl.BlockSpec((1, 64), lambda i: (0, 0)),
        ],
        out_specs=pl.BlockSpec((BT, 128), lambda i: (i, 0)),
        out_shape=jax.ShapeDtypeStruct((E8, 128), jnp.float32),
    )(x_arr, x_arr, fj8, w1p, b1d, w2d, b2d, w3d, b3d)

    scatter_k = pl.kernel(
        _scatter_body,
        out_type=jax.ShapeDtypeStruct((NC, ACC_R, 16), jnp.float32),
        mesh=mesh,
        compiler_params=sc_params,
        scratch_types=(
            pltpu.VMEM((CH, 16), jnp.float32),
            pltpu.VMEM((CH,), jnp.int32),
            pltpu.MemorySpace.VMEM_SHARED((ACC_R, 16), jnp.float32),
        ),
    )
    partials = scatter_k(kk8, seg_p, zeros_stripe)

    out = pl.pallas_call(
        _combine_body,
        grid=(N // BN,),
        in_specs=[
            pl.BlockSpec((1, BN, 16), lambda i: (0, i, 0)),
            pl.BlockSpec((1, BN, 16), lambda i: (1, i, 0)),
            pl.BlockSpec((BN, 1), lambda i: (i, 0)),
        ],
        out_specs=pl.BlockSpec((BN, 16), lambda i: (i, 0)),
        out_shape=jax.ShapeDtypeStruct((N, FCH), jnp.float32),
    )(partials, partials, cnt_f)
    return out
